# Initial kernel scaffold; baseline (speedup 1.0000x reference)
import jax
import jax.numpy as jnp
from jax.experimental import pallas as pl


def kernel(x, edge_index, batch, W0, b0, p0, W1, b1, p1, W2, b2, p2):
    raise NotImplementedError("write your pallas kernel here")


# speedup vs baseline: 2.2304x; 2.2304x over previous
import functools

import jax
import jax.numpy as jnp
from jax.experimental import pallas as pl
from jax.experimental.pallas import tpu as pltpu

N = 10000
E = 320000
H = 128
G = 64
RATIO = 0.5
BLK = 2000


def _y_body(x_ref, w_ref, deg_ref, y_ref):
    dis = jax.lax.rsqrt(deg_ref[...])
    y_ref[...] = jnp.dot(x_ref[...], w_ref[...],
                         preferred_element_type=jnp.float32) * dis


def _compute_y(x, W, deg):
    return pl.pallas_call(
        _y_body,
        grid=(N // BLK,),
        in_specs=[
            pl.BlockSpec((BLK, H), lambda i: (i, 0)),
            pl.BlockSpec((H, H), lambda i: (0, 0)),
            pl.BlockSpec((BLK, 1), lambda i: (i, 0)),
        ],
        out_specs=pl.BlockSpec((BLK, H), lambda i: (i, 0)),
        out_shape=jax.ShapeDtypeStruct((N, H), jnp.float32),
    )(x, W, deg.reshape(N, 1))


def _conv_body(agg_ref, y_ref, deg_ref, b_ref, kept_ref, p_ref, pn_ref,
               xl_ref, sc_ref):
    dis = jax.lax.rsqrt(deg_ref[...])
    out = dis * (agg_ref[...] + y_ref[...]) + b_ref[...]
    xl = jnp.where(kept_ref[...] != 0.0, jnp.maximum(out, 0.0), 0.0)
    xl_ref[...] = xl
    sc_ref[...] = jnp.tanh(
        jnp.dot(xl, p_ref[...], preferred_element_type=jnp.float32)
        / pn_ref[0, 0])


def _conv_epilogue(agg, y, deg, b, keptf, p, pn):
    return pl.pallas_call(
        _conv_body,
        grid=(N // BLK,),
        in_specs=[
            pl.BlockSpec((BLK, H), lambda i: (i, 0)),
            pl.BlockSpec((BLK, H), lambda i: (i, 0)),
            pl.BlockSpec((BLK, 1), lambda i: (i, 0)),
            pl.BlockSpec((1, H), lambda i: (0, 0)),
            pl.BlockSpec((BLK, 1), lambda i: (i, 0)),
            pl.BlockSpec((H, 1), lambda i: (0, 0)),
            pl.BlockSpec((1, 1), lambda i: (0, 0)),
        ],
        out_specs=[
            pl.BlockSpec((BLK, H), lambda i: (i, 0)),
            pl.BlockSpec((BLK, 1), lambda i: (i, 0)),
        ],
        out_shape=[
            jax.ShapeDtypeStruct((N, H), jnp.float32),
            jax.ShapeDtypeStruct((N, 1), jnp.float32),
        ],
    )(agg, y, deg.reshape(N, 1), b.reshape(1, H), keptf.reshape(N, 1),
      p.reshape(H, 1), pn.reshape(1, 1))


def _pool_rank(score, batch, kept, prev_rank):
    n = score.shape[0]
    key = batch.astype(score.dtype) * 4.0 - score
    order = jnp.lexsort((prev_rank, key))
    kept_s = kept[order].astype(jnp.int32)
    g_s = batch[order]
    cnts = jax.ops.segment_sum(kept.astype(jnp.int32), batch, num_segments=G)
    ks = jnp.ceil(RATIO * cnts.astype(jnp.float32)).astype(jnp.int32)
    kept_off = jnp.cumsum(cnts) - cnts
    new_off = jnp.cumsum(ks) - ks
    rank_within = jnp.cumsum(kept_s) - 1 - kept_off[g_s]
    sel_s = (kept_s == 1) & (rank_within < ks[g_s])
    pos_s = jnp.where(sel_s, new_off[g_s] + rank_within, n).astype(jnp.int32)
    sel = jnp.zeros((n,), jnp.bool_).at[order].set(sel_s)
    pos = jnp.full((n,), n, jnp.int32).at[order].set(pos_s)
    return sel, pos, ks


def _readout(x, pos, ks):
    n = x.shape[0]
    buf = jnp.zeros_like(x).at[pos].set(x, mode='drop')
    slot_g = jnp.searchsorted(jnp.cumsum(ks), jnp.arange(n, dtype=jnp.int32),
                              side='right').astype(jnp.int32)
    mx = jax.ops.segment_max(buf, slot_g, num_segments=G)
    sm = jax.ops.segment_sum(buf, slot_g, num_segments=G)
    cnt = jax.ops.segment_sum(jnp.ones((n,), x.dtype), slot_g, num_segments=G)
    mean = sm / jnp.maximum(cnt, 1.0)[:, None]
    return jnp.concatenate([mx, mean], axis=1)


def kernel(x, edge_index, batch, W0, b0, p0, W1, b1, p1, W2, b2, p2):
    src = edge_index[0].astype(jnp.int32)
    dst = edge_index[1].astype(jnp.int32)
    batch = batch.astype(jnp.int32)
    params = [(W0, b0, p0), (W1, b1, p1), (W2, b2, p2)]

    kept = jnp.ones((N,), jnp.bool_)
    prev_rank = jnp.arange(N, dtype=jnp.int32)
    out = None
    for W, b, p in params:
        keptf = kept.astype(jnp.float32)
        deg = jnp.ones((N,), jnp.float32).at[dst].add(keptf[src])
        y = _compute_y(x, W, deg)
        agg = jnp.zeros((N, H), jnp.float32).at[dst].add(y[src])
        xl, score2 = _conv_epilogue(agg, y, deg, b, keptf, p,
                                    jnp.linalg.norm(p))
        score = score2[:, 0]
        sel, pos, ks = _pool_rank(score, batch, kept, prev_rank)
        x = jnp.where(sel[:, None], xl * score[:, None], 0.0)
        r = _readout(x, pos, ks)
        out = r if out is None else out + r
        kept = sel
        prev_rank = pos
    return out


# speedup vs baseline: 2.9394x; 1.3178x over previous
import functools

import jax
import jax.numpy as jnp
from jax import lax
from jax.experimental import pallas as pl
from jax.experimental.pallas import tpu as pltpu
from jax.experimental.pallas import tpu_sc as plsc

N = 10000
E = 320000
H = 128
G = 64
RATIO = 0.5
BLK = 2000

_NC = 2
_NS = 16
_CHW = 80
_NCH = E // (_NC * _NS * _CHW)
_IBK = 25
_NP = 10240
_RPT = _NP // _NS
_ZB = 64


def _agg_body(y_hbm, src_hbm, dst_hbm, out_hbm, sidx, didx, rbuf, zbuf, sem,
              acc):
    c = lax.axis_index("c")
    s = lax.axis_index("s")

    if True:
        def zrow(i, _):
            for j in range(H // 16):
                zbuf[i, pl.ds(j * 16, 16)] = jnp.zeros((16,), jnp.float32)
            return 0
        lax.fori_loop(0, _ZB, zrow, 0, unroll=False)
        for k in range(_RPT // _ZB):
            pltpu.sync_copy(zbuf, acc.at[pl.ds(s * _RPT + k * _ZB, _ZB)])
        plsc.subcore_barrier()

        def block(b, _):
            pltpu.sync_copy(src_hbm.at[c, s, b], sidx)
            pltpu.sync_copy(dst_hbm.at[c, s, b], didx)

            def chunk(i, _):
                pltpu.async_copy(y_hbm.at[sidx.at[i]], rbuf, sem).wait()
                pltpu.sync_copy(rbuf, acc.at[didx.at[i]], add=True)
                return 0
            return lax.fori_loop(0, _IBK, chunk, 0, unroll=False)
        lax.fori_loop(0, _NCH // _IBK, block, 0, unroll=False)
        plsc.subcore_barrier()
        pltpu.sync_copy(acc.at[pl.ds(s * _RPT, _RPT)],
                        out_hbm.at[c, pl.ds(s * _RPT, _RPT)])


@functools.cache
def _sc_aggregate_fn():
    return pl.kernel(
        _agg_body,
        out_type=jax.ShapeDtypeStruct((_NC, _NP, H), jnp.float32),
        mesh=plsc.VectorSubcoreMesh(core_axis_name="c", subcore_axis_name="s"),
        scratch_types=[
            pltpu.VMEM((_IBK, _CHW), jnp.int32),
            pltpu.VMEM((_IBK, _CHW), jnp.int32),
            pltpu.VMEM((_CHW, H), jnp.float32),
            pltpu.VMEM((_ZB, H), jnp.float32),
            pltpu.SemaphoreType.DMA,
            pltpu.VMEM_SHARED((_NP, H), jnp.float32),
        ],
    )


def _y_body(x_ref, w_ref, deg_ref, y_ref):
    dis = jax.lax.rsqrt(deg_ref[...])
    y_ref[...] = jnp.dot(x_ref[...], w_ref[...],
                         preferred_element_type=jnp.float32) * dis


def _compute_y(x, W, deg):
    return pl.pallas_call(
        _y_body,
        grid=(N // BLK,),
        in_specs=[
            pl.BlockSpec((BLK, H), lambda i: (i, 0)),
            pl.BlockSpec((H, H), lambda i: (0, 0)),
            pl.BlockSpec((BLK, 1), lambda i: (i, 0)),
        ],
        out_specs=pl.BlockSpec((BLK, H), lambda i: (i, 0)),
        out_shape=jax.ShapeDtypeStruct((N, H), jnp.float32),
    )(x, W, deg.reshape(N, 1))


def _conv_body(agg_ref, y_ref, deg_ref, b_ref, kept_ref, p_ref, pn_ref,
               xl_ref, sc_ref):
    dis = jax.lax.rsqrt(deg_ref[...])
    out = dis * (agg_ref[...] + y_ref[...]) + b_ref[...]
    xl = jnp.where(kept_ref[...] != 0.0, jnp.maximum(out, 0.0), 0.0)
    xl_ref[...] = xl
    sc_ref[...] = jnp.tanh(
        jnp.dot(xl, p_ref[...], preferred_element_type=jnp.float32)
        / pn_ref[0, 0])


def _conv_epilogue(agg, y, deg, b, keptf, p, pn):
    return pl.pallas_call(
        _conv_body,
        grid=(N // BLK,),
        in_specs=[
            pl.BlockSpec((BLK, H), lambda i: (i, 0)),
            pl.BlockSpec((BLK, H), lambda i: (i, 0)),
            pl.BlockSpec((BLK, 1), lambda i: (i, 0)),
            pl.BlockSpec((1, H), lambda i: (0, 0)),
            pl.BlockSpec((BLK, 1), lambda i: (i, 0)),
            pl.BlockSpec((H, 1), lambda i: (0, 0)),
            pl.BlockSpec((1, 1), lambda i: (0, 0)),
        ],
        out_specs=[
            pl.BlockSpec((BLK, H), lambda i: (i, 0)),
            pl.BlockSpec((BLK, 1), lambda i: (i, 0)),
        ],
        out_shape=[
            jax.ShapeDtypeStruct((N, H), jnp.float32),
            jax.ShapeDtypeStruct((N, 1), jnp.float32),
        ],
    )(agg, y, deg.reshape(N, 1), b.reshape(1, H), keptf.reshape(N, 1),
      p.reshape(H, 1), pn.reshape(1, 1))


def _pool_rank(score, batch, kept, prev_rank):
    n = score.shape[0]
    key = batch.astype(score.dtype) * 4.0 - score
    order = jnp.lexsort((prev_rank, key))
    kept_s = kept[order].astype(jnp.int32)
    g_s = batch[order]
    cnts = jax.ops.segment_sum(kept.astype(jnp.int32), batch, num_segments=G)
    ks = jnp.ceil(RATIO * cnts.astype(jnp.float32)).astype(jnp.int32)
    kept_off = jnp.cumsum(cnts) - cnts
    new_off = jnp.cumsum(ks) - ks
    rank_within = jnp.cumsum(kept_s) - 1 - kept_off[g_s]
    sel_s = (kept_s == 1) & (rank_within < ks[g_s])
    pos_s = jnp.where(sel_s, new_off[g_s] + rank_within, n).astype(jnp.int32)
    sel = jnp.zeros((n,), jnp.bool_).at[order].set(sel_s)
    pos = jnp.full((n,), n, jnp.int32).at[order].set(pos_s)
    return sel, pos, ks


def _readout(x, pos, ks):
    n = x.shape[0]
    buf = jnp.zeros_like(x).at[pos].set(x, mode='drop')
    slot_g = jnp.searchsorted(jnp.cumsum(ks), jnp.arange(n, dtype=jnp.int32),
                              side='right').astype(jnp.int32)
    mx = jax.ops.segment_max(buf, slot_g, num_segments=G)
    sm = jax.ops.segment_sum(buf, slot_g, num_segments=G)
    cnt = jax.ops.segment_sum(jnp.ones((n,), x.dtype), slot_g, num_segments=G)
    mean = sm / jnp.maximum(cnt, 1.0)[:, None]
    return jnp.concatenate([mx, mean], axis=1)


def kernel(x, edge_index, batch, W0, b0, p0, W1, b1, p1, W2, b2, p2):
    src = edge_index[0].astype(jnp.int32)
    dst = edge_index[1].astype(jnp.int32)
    batch = batch.astype(jnp.int32)
    params = [(W0, b0, p0), (W1, b1, p1), (W2, b2, p2)]

    kept = jnp.ones((N,), jnp.bool_)
    prev_rank = jnp.arange(N, dtype=jnp.int32)
    out = None
    for W, b, p in params:
        keptf = kept.astype(jnp.float32)
        deg = jnp.ones((N,), jnp.float32).at[dst].add(keptf[src])
        y = _compute_y(x, W, deg)
        src4 = src.reshape(_NC, _NS, _NCH // _IBK, _IBK, _CHW)
        dst4 = dst.reshape(_NC, _NS, _NCH // _IBK, _IBK, _CHW)
        parts = _sc_aggregate_fn()(y, src4, dst4)
        agg = parts[0, :N] + parts[1, :N]
        xl, score2 = _conv_epilogue(agg, y, deg, b, keptf, p,
                                    jnp.linalg.norm(p))
        score = score2[:, 0]
        sel, pos, ks = _pool_rank(score, batch, kept, prev_rank)
        x = jnp.where(sel[:, None], xl * score[:, None], 0.0)
        r = _readout(x, pos, ks)
        out = r if out is None else out + r
        kept = sel
        prev_rank = pos
    return out


# speedup vs baseline: 7.1864x; 2.4449x over previous
import functools

import jax
import jax.numpy as jnp
from jax import lax
from jax.experimental import pallas as pl
from jax.experimental.pallas import tpu as pltpu
from jax.experimental.pallas import tpu_sc as plsc

N = 10000
E = 320000
H = 128
G = 64
RATIO = 0.5
BLK = 2000

_NC = 2
_NS = 16
_CHW = 80
_NCH = E // (_NC * _NS * _CHW)
_IBK = 25
_NP = 10240
_RPT = _NP // _NS
_ZB = 64


def _agg_body(y_hbm, src_hbm, dst_hbm, out_hbm, sidx, didx, rbuf, zbuf, sem,
              acc):
    c = lax.axis_index("c")
    s = lax.axis_index("s")

    if True:
        def zrow(i, _):
            for j in range(H // 16):
                zbuf[i, pl.ds(j * 16, 16)] = jnp.zeros((16,), jnp.float32)
            return 0
        lax.fori_loop(0, _ZB, zrow, 0, unroll=False)
        for k in range(_RPT // _ZB):
            pltpu.sync_copy(zbuf, acc.at[pl.ds(s * _RPT + k * _ZB, _ZB)])
        plsc.subcore_barrier()

        def block(b, _):
            pltpu.sync_copy(src_hbm.at[c, s, b], sidx)
            pltpu.sync_copy(dst_hbm.at[c, s, b], didx)

            def chunk(i, _):
                pltpu.async_copy(y_hbm.at[sidx.at[i]], rbuf, sem).wait()
                pltpu.sync_copy(rbuf, acc.at[didx.at[i]], add=True)
                return 0
            return lax.fori_loop(0, _IBK, chunk, 0, unroll=False)
        lax.fori_loop(0, _NCH // _IBK, block, 0, unroll=False)
        plsc.subcore_barrier()
        pltpu.sync_copy(acc.at[pl.ds(s * _RPT, _RPT)],
                        out_hbm.at[c, pl.ds(s * _RPT, _RPT)])


_DW = 16


def _hist_body(keptf_hbm, src_hbm, dst_hbm, out_hbm, sidx, didx, vbuf, zbuf,
               sem, acc):
    c = lax.axis_index("c")
    s = lax.axis_index("s")

    for i in range(_RPT // 16):
        zbuf[pl.ds(i * 16, 16)] = jnp.zeros((16,), jnp.float32)
    pltpu.sync_copy(zbuf, acc.at[pl.ds(s * _RPT, _RPT)])
    plsc.subcore_barrier()

    def block(b, _):
        pltpu.sync_copy(src_hbm.at[c, s, b], sidx)
        pltpu.sync_copy(dst_hbm.at[c, s, b], didx)

        def chunk(i, _):
            pltpu.async_copy(keptf_hbm.at[sidx.at[i]], vbuf, sem).wait()
            pltpu.sync_copy(vbuf, acc.at[didx.at[i]], add=True)
            return 0
        return lax.fori_loop(0, _IBK, chunk, 0, unroll=False)
    lax.fori_loop(0, _NCH // _IBK, block, 0, unroll=False)
    plsc.subcore_barrier()
    pltpu.sync_copy(acc.at[pl.ds(s * _RPT, _RPT)],
                    out_hbm.at[pl.ds(c * _NP + s * _RPT, _RPT)])


@functools.cache
def _sc_hist_fn():
    return pl.kernel(
        _hist_body,
        out_type=jax.ShapeDtypeStruct((_NC * _NP,), jnp.float32),
        mesh=plsc.VectorSubcoreMesh(core_axis_name="c", subcore_axis_name="s"),
        scratch_types=[
            pltpu.VMEM((_IBK, _CHW), jnp.int32),
            pltpu.VMEM((_IBK, _CHW), jnp.int32),
            pltpu.VMEM((_CHW,), jnp.float32),
            pltpu.VMEM((_RPT,), jnp.float32),
            pltpu.SemaphoreType.DMA,
            pltpu.VMEM_SHARED((_NP,), jnp.float32),
        ],
    )


@functools.cache
def _sc_aggregate_fn():
    return pl.kernel(
        _agg_body,
        out_type=jax.ShapeDtypeStruct((_NC, _NP, H), jnp.float32),
        mesh=plsc.VectorSubcoreMesh(core_axis_name="c", subcore_axis_name="s"),
        scratch_types=[
            pltpu.VMEM((_IBK, _CHW), jnp.int32),
            pltpu.VMEM((_IBK, _CHW), jnp.int32),
            pltpu.VMEM((_CHW, H), jnp.float32),
            pltpu.VMEM((_ZB, H), jnp.float32),
            pltpu.SemaphoreType.DMA,
            pltpu.VMEM_SHARED((_NP, H), jnp.float32),
        ],
    )


def _y_body(x_ref, w_ref, deg_ref, y_ref):
    dis = jax.lax.rsqrt(deg_ref[...])
    y_ref[...] = jnp.dot(x_ref[...], w_ref[...],
                         preferred_element_type=jnp.float32) * dis


def _compute_y(x, W, deg):
    return pl.pallas_call(
        _y_body,
        grid=(N // BLK,),
        in_specs=[
            pl.BlockSpec((BLK, H), lambda i: (i, 0)),
            pl.BlockSpec((H, H), lambda i: (0, 0)),
            pl.BlockSpec((BLK, 1), lambda i: (i, 0)),
        ],
        out_specs=pl.BlockSpec((BLK, H), lambda i: (i, 0)),
        out_shape=jax.ShapeDtypeStruct((N, H), jnp.float32),
    )(x, W, deg.reshape(N, 1))


def _conv_body(agg_ref, y_ref, deg_ref, b_ref, kept_ref, p_ref, pn_ref,
               xl_ref, sc_ref):
    dis = jax.lax.rsqrt(deg_ref[...])
    out = dis * (agg_ref[...] + y_ref[...]) + b_ref[...]
    xl = jnp.where(kept_ref[...] != 0.0, jnp.maximum(out, 0.0), 0.0)
    xl_ref[...] = xl
    sc_ref[...] = jnp.tanh(
        jnp.dot(xl, p_ref[...], preferred_element_type=jnp.float32)
        / pn_ref[0, 0])


def _conv_epilogue(agg, y, deg, b, keptf, p, pn):
    return pl.pallas_call(
        _conv_body,
        grid=(N // BLK,),
        in_specs=[
            pl.BlockSpec((BLK, H), lambda i: (i, 0)),
            pl.BlockSpec((BLK, H), lambda i: (i, 0)),
            pl.BlockSpec((BLK, 1), lambda i: (i, 0)),
            pl.BlockSpec((1, H), lambda i: (0, 0)),
            pl.BlockSpec((BLK, 1), lambda i: (i, 0)),
            pl.BlockSpec((H, 1), lambda i: (0, 0)),
            pl.BlockSpec((1, 1), lambda i: (0, 0)),
        ],
        out_specs=[
            pl.BlockSpec((BLK, H), lambda i: (i, 0)),
            pl.BlockSpec((BLK, 1), lambda i: (i, 0)),
        ],
        out_shape=[
            jax.ShapeDtypeStruct((N, H), jnp.float32),
            jax.ShapeDtypeStruct((N, 1), jnp.float32),
        ],
    )(agg, y, deg.reshape(N, 1), b.reshape(1, H), keptf.reshape(N, 1),
      p.reshape(H, 1), pn.reshape(1, 1))


def _pool_rank(score, batch, kept, prev_rank):
    n = score.shape[0]
    key = batch.astype(score.dtype) * 4.0 - score
    order = jnp.lexsort((prev_rank, key))
    kept_s = kept[order].astype(jnp.int32)
    g_s = batch[order]
    cnts = jax.ops.segment_sum(kept.astype(jnp.int32), batch, num_segments=G)
    ks = jnp.ceil(RATIO * cnts.astype(jnp.float32)).astype(jnp.int32)
    kept_off = jnp.cumsum(cnts) - cnts
    new_off = jnp.cumsum(ks) - ks
    rank_within = jnp.cumsum(kept_s) - 1 - kept_off[g_s]
    sel_s = (kept_s == 1) & (rank_within < ks[g_s])
    pos_s = jnp.where(sel_s, new_off[g_s] + rank_within, n).astype(jnp.int32)
    sel = jnp.zeros((n,), jnp.bool_).at[order].set(sel_s)
    pos = jnp.full((n,), n, jnp.int32).at[order].set(pos_s)
    return sel, pos, ks


def _readout(x, pos, ks):
    n = x.shape[0]
    buf = jnp.zeros_like(x).at[pos].set(x, mode='drop')
    slot_g = jnp.searchsorted(jnp.cumsum(ks), jnp.arange(n, dtype=jnp.int32),
                              side='right').astype(jnp.int32)
    mx = jax.ops.segment_max(buf, slot_g, num_segments=G)
    sm = jax.ops.segment_sum(buf, slot_g, num_segments=G)
    cnt = jax.ops.segment_sum(jnp.ones((n,), x.dtype), slot_g, num_segments=G)
    mean = sm / jnp.maximum(cnt, 1.0)[:, None]
    return jnp.concatenate([mx, mean], axis=1)


def kernel(x, edge_index, batch, W0, b0, p0, W1, b1, p1, W2, b2, p2):
    src = edge_index[0].astype(jnp.int32)
    dst = edge_index[1].astype(jnp.int32)
    batch = batch.astype(jnp.int32)
    params = [(W0, b0, p0), (W1, b1, p1), (W2, b2, p2)]

    src4 = src.reshape(_NC, _NS, _NCH // _IBK, _IBK, _CHW)
    dst4 = dst.reshape(_NC, _NS, _NCH // _IBK, _IBK, _CHW)

    kept = jnp.ones((N,), jnp.bool_)
    prev_rank = jnp.arange(N, dtype=jnp.int32)
    out = None
    for W, b, p in params:
        keptf = kept.astype(jnp.float32)
        hparts = _sc_hist_fn()(keptf, src4, dst4)
        deg = 1.0 + hparts[:N] + hparts[_NP:_NP + N]
        y = _compute_y(x, W, deg)
        parts = _sc_aggregate_fn()(y, src4, dst4)
        agg = parts[0, :N] + parts[1, :N]
        xl, score2 = _conv_epilogue(agg, y, deg, b, keptf, p,
                                    jnp.linalg.norm(p))
        score = score2[:, 0]
        sel, pos, ks = _pool_rank(score, batch, kept, prev_rank)
        x = jnp.where(sel[:, None], xl * score[:, None], 0.0)
        r = _readout(x, pos, ks)
        out = r if out is None else out + r
        kept = sel
        prev_rank = pos
    return out


# speedup vs baseline: 11.8812x; 1.6533x over previous
import functools

import jax
import jax.numpy as jnp
from jax import lax
from jax.experimental import pallas as pl
from jax.experimental.pallas import tpu as pltpu
from jax.experimental.pallas import tpu_sc as plsc

N = 10000
E = 320000
H = 128
G = 64
RATIO = 0.5
BLK = 2000
BLKE = 2048

_NC = 2
_NS = 16
_CHW = 80
_NCH = E // (_NC * _NS * _CHW)
_IBK = 25
_NP = 10240
_RPT = _NP // _NS
_ZB = 64
_BIG = 1e30
_RMAX = 10000
_RZERO = 10016
_DUMP = 10200


def _hist_body(keptf_hbm, src_hbm, dst_hbm, out_hbm, sidx, didx, vbuf, zbuf,
               sem, acc):
    c = lax.axis_index("c")
    s = lax.axis_index("s")

    for i in range(_RPT // 16):
        zbuf[pl.ds(i * 16, 16)] = jnp.zeros((16,), jnp.float32)
    pltpu.sync_copy(zbuf, acc.at[pl.ds(s * _RPT, _RPT)])
    plsc.subcore_barrier()

    def block(b, _):
        pltpu.sync_copy(src_hbm.at[c, s, b], sidx)
        pltpu.sync_copy(dst_hbm.at[c, s, b], didx)

        def chunk(i, _):
            pltpu.async_copy(keptf_hbm.at[sidx.at[i]], vbuf, sem).wait()
            pltpu.sync_copy(vbuf, acc.at[didx.at[i]], add=True)
            return 0
        return lax.fori_loop(0, _IBK, chunk, 0, unroll=False)
    lax.fori_loop(0, _NCH // _IBK, block, 0, unroll=False)
    plsc.subcore_barrier()
    pltpu.sync_copy(acc.at[pl.ds(s * _RPT, _RPT)],
                    out_hbm.at[pl.ds(c * _NP + s * _RPT, _RPT)])


@functools.cache
def _sc_hist_fn():
    return pl.kernel(
        _hist_body,
        out_type=jax.ShapeDtypeStruct((_NC * _NP,), jnp.float32),
        mesh=plsc.VectorSubcoreMesh(core_axis_name="c", subcore_axis_name="s"),
        scratch_types=[
            pltpu.VMEM((_IBK, _CHW), jnp.int32),
            pltpu.VMEM((_IBK, _CHW), jnp.int32),
            pltpu.VMEM((_CHW,), jnp.float32),
            pltpu.VMEM((_RPT,), jnp.float32),
            pltpu.SemaphoreType.DMA,
            pltpu.VMEM_SHARED((_NP,), jnp.float32),
        ],
    )


def _agg_body(y_hbm, src_hbm, dst_hbm, out_hbm, sidx, didx, rbuf, zbuf, sem,
              acc):
    c = lax.axis_index("c")
    s = lax.axis_index("s")

    def zrow(i, _):
        for j in range(H // 16):
            zbuf[i, pl.ds(j * 16, 16)] = jnp.zeros((16,), jnp.float32)
        return 0
    lax.fori_loop(0, _ZB, zrow, 0, unroll=False)
    for k in range(_RPT // _ZB):
        pltpu.sync_copy(zbuf, acc.at[pl.ds(s * _RPT + k * _ZB, _ZB)])
    plsc.subcore_barrier()

    def block(b, _):
        pltpu.sync_copy(src_hbm.at[c, s, b], sidx)
        pltpu.sync_copy(dst_hbm.at[c, s, b], didx)

        def chunk(i, _):
            pltpu.async_copy(y_hbm.at[sidx.at[i]], rbuf, sem).wait()
            pltpu.sync_copy(rbuf, acc.at[didx.at[i]], add=True)
            return 0
        return lax.fori_loop(0, _IBK, chunk, 0, unroll=False)
    lax.fori_loop(0, _NCH // _IBK, block, 0, unroll=False)
    plsc.subcore_barrier()
    pltpu.sync_copy(acc.at[pl.ds(s * _RPT, _RPT)],
                    out_hbm.at[c, pl.ds(s * _RPT, _RPT)])


@functools.cache
def _sc_aggregate_fn():
    return pl.kernel(
        _agg_body,
        out_type=jax.ShapeDtypeStruct((_NC, _NP, H), jnp.float32),
        mesh=plsc.VectorSubcoreMesh(core_axis_name="c", subcore_axis_name="s"),
        scratch_types=[
            pltpu.VMEM((_IBK, _CHW), jnp.int32),
            pltpu.VMEM((_IBK, _CHW), jnp.int32),
            pltpu.VMEM((_CHW, H), jnp.float32),
            pltpu.VMEM((_ZB, H), jnp.float32),
            pltpu.SemaphoreType.DMA,
            pltpu.VMEM_SHARED((_NP, H), jnp.float32),
        ],
    )


def _pool_body(order_hbm, ks_s_hbm, cs_hbm, xs_hbm, tab_hbm,
               sel_hbm, perm_hbm, ms_hbm,
               obuf, kbuf, csb, tbuf, wbuf, pbuf, m1buf, m2buf, rbufA, rbufB,
               idx16, out16, sem, tab):
    w = lax.axis_index("c") * _NS + lax.axis_index("s")
    iota16 = lax.iota(jnp.int32, 16)
    zero16 = jnp.zeros((16,), jnp.float32)
    nbig16 = jnp.full((16,), -_BIG, jnp.float32)

    pltpu.sync_copy(tab_hbm, tab)

    def sread(i):
        return tab[pl.ds(i, 16)][0]

    total = sread(4 * 80 + G).astype(jnp.int32)

    for gl in range(2):
        g = w * 2 + gl
        off = sread(g).astype(jnp.int32)
        off1 = sread(g + 1).astype(jnp.int32)
        ko = sread(80 + g).astype(jnp.int32)
        koff = sread(2 * 80 + g)
        ksgf = sread(3 * 80 + g)
        tail0 = total + (off - ko)
        base = (off // 8) * 8
        ntrip = (off1 - base + _CHW - 1) // _CHW

        for j in range(H // 16):
            out16[0, pl.ds(j * 16, 16)] = nbig16
            out16[1, pl.ds(j * 16, 16)] = zero16

        def chunk2(t, _):
            cbase = base + t * _CHW
            pltpu.sync_copy(order_hbm.at[pl.ds(cbase, _CHW)], obuf)
            pltpu.sync_copy(ks_s_hbm.at[pl.ds(cbase, _CHW)], kbuf)
            pltpu.sync_copy(cs_hbm.at[pl.ds(cbase, _CHW)], csb)
            for j in range(_CHW // 16):
                posv = cbase + j * 16 + iota16
                mask = (posv >= off) & (posv < off1)
                kf = kbuf[pl.ds(j * 16, 16)]
                rank = csb[pl.ds(j * 16, 16)] - koff
                selv = mask & (kf > 0.0) & (rank <= ksgf)
                ov = obuf[pl.ds(j * 16, 16)]
                wbuf[pl.ds(j * 16, 16)] = jnp.where(selv, 1.0, 0.0)
                tbuf[pl.ds(j * 16, 16)] = jnp.where(mask, ov, _DUMP)
                inseg = (posv - off + 1).astype(jnp.float32)
                nonsel = inseg - jnp.minimum(jnp.maximum(rank, 0.0), ksgf)
                slot = jnp.where(
                    selv, ko + rank.astype(jnp.int32) - 1,
                    tail0 + nonsel.astype(jnp.int32) - 1)
                pbuf[pl.ds(j * 16, 16)] = jnp.where(mask, slot, _DUMP)
                m1buf[pl.ds(j * 16, 16)] = jnp.where(
                    selv, ov, _RMAX + (iota16 & 7))
                m2buf[pl.ds(j * 16, 16)] = jnp.where(
                    selv, ov, _RZERO + (iota16 & 7))
            pltpu.sync_copy(wbuf, sel_hbm.at[tbuf])
            pltpu.sync_copy(obuf, perm_hbm.at[pbuf])
            pltpu.async_copy(xs_hbm.at[m1buf], rbufA, sem).wait()
            pltpu.async_copy(xs_hbm.at[m2buf], rbufB, sem).wait()

            def row(r, _):
                for j in range(H // 16):
                    a = rbufA[r, pl.ds(j * 16, 16)]
                    bv = rbufB[r, pl.ds(j * 16, 16)]
                    out16[0, pl.ds(j * 16, 16)] = jnp.maximum(
                        out16[0, pl.ds(j * 16, 16)], a)
                    out16[1, pl.ds(j * 16, 16)] = (
                        out16[1, pl.ds(j * 16, 16)] + bv)
                return 0
            return lax.fori_loop(0, _CHW, row, 0, unroll=False)
        lax.fori_loop(0, ntrip, chunk2, 0, unroll=False)

        idx16[pl.ds(0, 16)] = jnp.where(
            iota16 == 0, g,
            jnp.where(iota16 == 1, 72 + g, 136 + (iota16 & 7)))
        pltpu.sync_copy(out16, ms_hbm.at[idx16])


@functools.cache
def _sc_pool_fn():
    return pl.kernel(
        _pool_body,
        out_type=(
            jax.ShapeDtypeStruct((_NP,), jnp.float32),
            jax.ShapeDtypeStruct((_NP,), jnp.int32),
            jax.ShapeDtypeStruct((144, H), jnp.float32),
        ),
        mesh=plsc.VectorSubcoreMesh(core_axis_name="c", subcore_axis_name="s"),
        scratch_types=[
            pltpu.VMEM((_CHW,), jnp.int32),
            pltpu.VMEM((_CHW,), jnp.float32),
            pltpu.VMEM((_CHW,), jnp.float32),
            pltpu.VMEM((_CHW,), jnp.int32),
            pltpu.VMEM((_CHW,), jnp.float32),
            pltpu.VMEM((_CHW,), jnp.int32),
            pltpu.VMEM((_CHW,), jnp.int32),
            pltpu.VMEM((_CHW,), jnp.int32),
            pltpu.VMEM((_CHW, H), jnp.float32),
            pltpu.VMEM((_CHW, H), jnp.float32),
            pltpu.VMEM((16,), jnp.int32),
            pltpu.VMEM((16, H), jnp.float32),
            pltpu.SemaphoreType.DMA,
            pltpu.VMEM((400,), jnp.float32),
        ],
    )


def _y_body(x_ref, sel_ref, w_ref, deg_ref, y_ref):
    dis = jax.lax.rsqrt(deg_ref[...])
    y_ref[...] = jnp.dot(x_ref[...] * sel_ref[...], w_ref[...],
                         preferred_element_type=jnp.float32) * dis


def _compute_y(x, sel, W, deg):
    return pl.pallas_call(
        _y_body,
        grid=(N // BLK,),
        in_specs=[
            pl.BlockSpec((BLK, H), lambda i: (i, 0)),
            pl.BlockSpec((BLK, 1), lambda i: (i, 0)),
            pl.BlockSpec((H, H), lambda i: (0, 0)),
            pl.BlockSpec((BLK, 1), lambda i: (i, 0)),
        ],
        out_specs=pl.BlockSpec((BLK, H), lambda i: (i, 0)),
        out_shape=jax.ShapeDtypeStruct((N, H), jnp.float32),
    )(x, sel, W, deg.reshape(N, 1))


def _conv_body(agg_ref, y_ref, deg_ref, b_ref, kept_ref, p_ref, pn_ref,
               xs_ref, sc_ref):
    i = pl.program_id(0)
    rows = i * BLKE + jax.lax.broadcasted_iota(jnp.int32, (BLKE, 1), 0)
    dis = jax.lax.rsqrt(deg_ref[...])
    conv = dis * (agg_ref[0] + agg_ref[1] + y_ref[...]) + b_ref[...]
    xl = jnp.where(kept_ref[...] != 0.0, jnp.maximum(conv, 0.0), 0.0)
    score = jnp.tanh(
        jnp.dot(xl, p_ref[...], preferred_element_type=jnp.float32)
        / pn_ref[0, 0])
    sc_ref[...] = score
    xs = xl * score
    pad = jnp.where(rows < _RZERO, -_BIG, 0.0)
    xs_ref[...] = jnp.where(rows < N, xs, pad)


def _conv_epilogue(parts, y, deg, b, keptf, p, pn):
    return pl.pallas_call(
        _conv_body,
        grid=(_NP // BLKE,),
        in_specs=[
            pl.BlockSpec((_NC, BLKE, H), lambda i: (0, i, 0)),
            pl.BlockSpec((BLKE, H), lambda i: (i, 0)),
            pl.BlockSpec((BLKE, 1), lambda i: (i, 0)),
            pl.BlockSpec((1, H), lambda i: (0, 0)),
            pl.BlockSpec((BLKE, 1), lambda i: (i, 0)),
            pl.BlockSpec((H, 1), lambda i: (0, 0)),
            pl.BlockSpec((1, 1), lambda i: (0, 0)),
        ],
        out_specs=[
            pl.BlockSpec((BLKE, H), lambda i: (i, 0)),
            pl.BlockSpec((BLKE, 1), lambda i: (i, 0)),
        ],
        out_shape=[
            jax.ShapeDtypeStruct((_NP, H), jnp.float32),
            jax.ShapeDtypeStruct((_NP, 1), jnp.float32),
        ],
    )(parts, y, deg.reshape(N, 1), b.reshape(1, H), keptf.reshape(N, 1),
      p.reshape(H, 1), pn.reshape(1, 1))


def kernel(x, edge_index, batch, W0, b0, p0, W1, b1, p1, W2, b2, p2):
    src = edge_index[0].astype(jnp.int32)
    dst = edge_index[1].astype(jnp.int32)
    batch = batch.astype(jnp.int32)
    params = [(W0, b0, p0), (W1, b1, p1), (W2, b2, p2)]

    src4 = src.reshape(_NC, _NS, _NCH // _IBK, _IBK, _CHW)
    dst4 = dst.reshape(_NC, _NS, _NCH // _IBK, _IBK, _CHW)

    node_off = jnp.searchsorted(
        batch, jnp.arange(G + 1, dtype=jnp.int32)).astype(jnp.int32)

    keptf = jnp.ones((N,), jnp.float32)
    selcol = jnp.ones((N, 1), jnp.float32)
    perm = jnp.arange(N, dtype=jnp.int32)
    cnts = node_off[1:] - node_off[:-1]
    xin = x
    out = None
    for W, b, p in params:
        hparts = _sc_hist_fn()(keptf, src4, dst4)
        deg = 1.0 + hparts[:N] + hparts[_NP:_NP + N]
        y = _compute_y(xin, selcol, W, deg)
        parts = _sc_aggregate_fn()(y, src4, dst4)
        xs_pad, score2 = _conv_epilogue(parts, y, deg, b, keptf, p,
                                        jnp.linalg.norm(p))
        score = score2[:N, 0]

        key = batch.astype(jnp.float32) * 4.0 - score
        kp = key[perm]
        _, order = lax.sort((kp, perm), num_keys=1, is_stable=True)
        order_pad = jnp.pad(order, (0, _NP - N))
        kept_s = keptf[order]
        cs = jnp.cumsum(kept_s)
        kept_s_pad = jnp.pad(kept_s, (0, _NP - N))
        cs_pad = jnp.pad(cs, (0, _NP - N))

        ks = jnp.ceil(RATIO * cnts.astype(jnp.float32)).astype(jnp.int32)
        new_off = jnp.cumsum(ks) - ks
        kept_off = jnp.cumsum(cnts) - cnts
        tab = jnp.zeros((400,), jnp.float32)
        tab = tab.at[:G + 1].set(node_off.astype(jnp.float32))
        tab = tab.at[80:80 + G].set(new_off.astype(jnp.float32))
        tab = tab.at[160:160 + G].set(kept_off.astype(jnp.float32))
        tab = tab.at[240:240 + G].set(ks.astype(jnp.float32))
        tab = tab.at[384].set(jnp.sum(ks).astype(jnp.float32))

        sel, perm_next, ms = _sc_pool_fn()(
            order_pad, kept_s_pad, cs_pad, xs_pad, tab)

        ksf = ks.astype(jnp.float32)
        mean = ms[72:72 + G] / jnp.maximum(ksf, 1.0)[:, None]
        r = jnp.concatenate([ms[:G], mean], axis=1)
        out = r if out is None else out + r

        keptf = sel[:N]
        selcol = keptf.reshape(N, 1)
        xin = xs_pad
        perm = perm_next[:N]
        cnts = ks
    return out


# speedup vs baseline: 13.3011x; 1.1195x over previous
import functools

import jax
import jax.numpy as jnp
from jax import lax
from jax.experimental import pallas as pl
from jax.experimental.pallas import tpu as pltpu
from jax.experimental.pallas import tpu_sc as plsc

N = 10000
E = 320000
H = 128
G = 64
RATIO = 0.5
BLK = 2000
BLKE = 2048

_NC = 2
_NS = 16
_CHW = 80
_NCH = E // (_NC * _NS * _CHW)
_IBK = 25
_NP = 10240
_RPT = _NP // _NS
_ZB = 64
_BIG = 1e30
_RMAX = 10000
_RZERO = 10016
_DUMP = 10200


def _hist_body(keptf_hbm, src_hbm, dst_hbm, out_hbm, sidx, didx, vbuf, zbuf,
               sem, acc):
    c = lax.axis_index("c")
    s = lax.axis_index("s")

    for i in range(_RPT // 16):
        zbuf[pl.ds(i * 16, 16)] = jnp.zeros((16,), jnp.float32)
    pltpu.sync_copy(zbuf, acc.at[pl.ds(s * _RPT, _RPT)])
    plsc.subcore_barrier()

    def block(b, _):
        pltpu.sync_copy(src_hbm.at[c, s, b], sidx)
        pltpu.sync_copy(dst_hbm.at[c, s, b], didx)

        def chunk(i, _):
            pltpu.async_copy(keptf_hbm.at[sidx.at[i]], vbuf, sem).wait()
            pltpu.sync_copy(vbuf, acc.at[didx.at[i]], add=True)
            return 0
        return lax.fori_loop(0, _IBK, chunk, 0, unroll=False)
    lax.fori_loop(0, _NCH // _IBK, block, 0, unroll=False)
    plsc.subcore_barrier()
    pltpu.sync_copy(acc.at[pl.ds(s * _RPT, _RPT)],
                    out_hbm.at[pl.ds(c * _NP + s * _RPT, _RPT)])


@functools.cache
def _sc_hist_fn():
    return pl.kernel(
        _hist_body,
        out_type=jax.ShapeDtypeStruct((_NC * _NP,), jnp.float32),
        mesh=plsc.VectorSubcoreMesh(core_axis_name="c", subcore_axis_name="s"),
        scratch_types=[
            pltpu.VMEM((_IBK, _CHW), jnp.int32),
            pltpu.VMEM((_IBK, _CHW), jnp.int32),
            pltpu.VMEM((_CHW,), jnp.float32),
            pltpu.VMEM((_RPT,), jnp.float32),
            pltpu.SemaphoreType.DMA,
            pltpu.VMEM_SHARED((_NP,), jnp.float32),
        ],
    )


def _agg_body(y_hbm, src_hbm, dst_hbm, out_hbm, sidx, didx, rbuf, zbuf, sem,
              semB, acc):
    c = lax.axis_index("c")
    s = lax.axis_index("s")

    def zrow(i, _):
        for j in range(H // 16):
            zbuf[i, pl.ds(j * 16, 16)] = jnp.zeros((16,), jnp.float32)
        return 0
    lax.fori_loop(0, _ZB, zrow, 0, unroll=False)
    for k in range(_RPT // _ZB):
        pltpu.sync_copy(zbuf, acc.at[pl.ds(s * _RPT + k * _ZB, _ZB)])
    plsc.subcore_barrier()

    rb0 = rbuf.at[0]
    rb1 = rbuf.at[1]

    def block(b, _):
        pltpu.sync_copy(src_hbm.at[c, s, b], sidx)
        pltpu.sync_copy(dst_hbm.at[c, s, b], didx)
        pltpu.async_copy(y_hbm.at[sidx.at[0]], rb0, sem)

        def pair(t, _):
            i = 2 * t
            pltpu.async_copy(y_hbm.at[sidx.at[i + 1]], rb1, semB)
            pltpu.make_async_copy(y_hbm.at[sidx.at[i]], rb0, sem).wait()
            pltpu.sync_copy(rb0, acc.at[didx.at[i]], add=True)
            pltpu.async_copy(y_hbm.at[sidx.at[i + 2]], rb0, sem)
            pltpu.make_async_copy(y_hbm.at[sidx.at[i + 1]], rb1, semB).wait()
            pltpu.sync_copy(rb1, acc.at[didx.at[i + 1]], add=True)
            return 0
        lax.fori_loop(0, (_IBK - 1) // 2, pair, 0, unroll=False)
        pltpu.make_async_copy(y_hbm.at[sidx.at[_IBK - 1]], rb0, sem).wait()
        pltpu.sync_copy(rb0, acc.at[didx.at[_IBK - 1]], add=True)
        return 0
    lax.fori_loop(0, _NCH // _IBK, block, 0, unroll=False)
    plsc.subcore_barrier()
    pltpu.sync_copy(acc.at[pl.ds(s * _RPT, _RPT)],
                    out_hbm.at[c, pl.ds(s * _RPT, _RPT)])


@functools.cache
def _sc_aggregate_fn():
    return pl.kernel(
        _agg_body,
        out_type=jax.ShapeDtypeStruct((_NC, _NP, H), jnp.float32),
        mesh=plsc.VectorSubcoreMesh(core_axis_name="c", subcore_axis_name="s"),
        scratch_types=[
            pltpu.VMEM((_IBK, _CHW), jnp.int32),
            pltpu.VMEM((_IBK, _CHW), jnp.int32),
            pltpu.VMEM((2, _CHW, H), jnp.float32),
            pltpu.VMEM((_ZB, H), jnp.float32),
            pltpu.SemaphoreType.DMA,
            pltpu.SemaphoreType.DMA,
            pltpu.VMEM_SHARED((_NP, H), jnp.float32),
        ],
    )


def _pool_body(order_hbm, ks_s_hbm, cs_hbm, xs_hbm, tab_hbm,
               sel_hbm, perm_hbm, ms_hbm,
               obuf, kbuf, csb, tbuf, wbuf, pbuf, m1buf, m2buf, rbufA, rbufB,
               idx16, out16, sem, semB, semS, semP, tab):
    w = lax.axis_index("c") * _NS + lax.axis_index("s")
    iota16 = lax.iota(jnp.int32, 16)
    zero16 = jnp.zeros((16,), jnp.float32)
    nbig16 = jnp.full((16,), -_BIG, jnp.float32)

    pltpu.sync_copy(tab_hbm, tab)

    def sread(i):
        return tab[pl.ds(i, 16)][0]

    total = sread(4 * 80 + G).astype(jnp.int32)

    for gl in range(2):
        g = w * 2 + gl
        off = sread(g).astype(jnp.int32)
        off1 = sread(g + 1).astype(jnp.int32)
        ko = sread(80 + g).astype(jnp.int32)
        koff = sread(2 * 80 + g)
        ksgf = sread(3 * 80 + g)
        tail0 = total + (off - ko)
        base = (off // 8) * 8
        ntrip = (off1 - base + _CHW - 1) // _CHW

        def chunk2(t, accs):
            cbase = base + t * _CHW
            pltpu.sync_copy(order_hbm.at[pl.ds(cbase, _CHW)], obuf)
            pltpu.sync_copy(ks_s_hbm.at[pl.ds(cbase, _CHW)], kbuf)
            pltpu.sync_copy(cs_hbm.at[pl.ds(cbase, _CHW)], csb)
            for j in range(_CHW // 16):
                posv = cbase + j * 16 + iota16
                mask = (posv >= off) & (posv < off1)
                kf = kbuf[pl.ds(j * 16, 16)]
                rank = csb[pl.ds(j * 16, 16)] - koff
                selv = mask & (kf > 0.0) & (rank <= ksgf)
                ov = obuf[pl.ds(j * 16, 16)]
                wbuf[pl.ds(j * 16, 16)] = jnp.where(selv, 1.0, 0.0)
                tbuf[pl.ds(j * 16, 16)] = jnp.where(mask, ov, _DUMP)
                inseg = (posv - off + 1).astype(jnp.float32)
                nonsel = inseg - jnp.minimum(jnp.maximum(rank, 0.0), ksgf)
                slot = jnp.where(
                    selv, ko + rank.astype(jnp.int32) - 1,
                    tail0 + nonsel.astype(jnp.int32) - 1)
                pbuf[pl.ds(j * 16, 16)] = jnp.where(mask, slot, _DUMP)
                m1buf[pl.ds(j * 16, 16)] = jnp.where(
                    selv, ov, _RMAX + (iota16 & 7))
                m2buf[pl.ds(j * 16, 16)] = jnp.where(
                    selv, ov, _RZERO + (iota16 & 7))
            cpS = pltpu.async_copy(wbuf, sel_hbm.at[tbuf], semS)
            cpP = pltpu.async_copy(obuf, perm_hbm.at[pbuf], semP)
            cpA = pltpu.async_copy(xs_hbm.at[m1buf], rbufA, sem)
            cpB = pltpu.async_copy(xs_hbm.at[m2buf], rbufB, semB)
            cpA.wait()
            cpB.wait()

            def row(r, a):
                mx = tuple(
                    jnp.maximum(a[j], rbufA[r, pl.ds(j * 16, 16)])
                    for j in range(8))
                sm = tuple(
                    a[8 + j] + rbufB[r, pl.ds(j * 16, 16)]
                    for j in range(8))
                return mx + sm
            accs = lax.fori_loop(0, _CHW, row, accs, unroll=False)
            cpS.wait()
            cpP.wait()
            return accs
        accs0 = tuple([nbig16] * 8 + [zero16] * 8)
        accs = lax.fori_loop(0, ntrip, chunk2, accs0, unroll=False)

        for j in range(H // 16):
            out16[0, pl.ds(j * 16, 16)] = accs[j]
            out16[1, pl.ds(j * 16, 16)] = accs[8 + j]
        idx16[pl.ds(0, 16)] = jnp.where(
            iota16 == 0, g,
            jnp.where(iota16 == 1, 72 + g, 136 + (iota16 & 7)))
        pltpu.sync_copy(out16, ms_hbm.at[idx16])


@functools.cache
def _sc_pool_fn():
    return pl.kernel(
        _pool_body,
        out_type=(
            jax.ShapeDtypeStruct((_NP,), jnp.float32),
            jax.ShapeDtypeStruct((_NP,), jnp.int32),
            jax.ShapeDtypeStruct((144, H), jnp.float32),
        ),
        mesh=plsc.VectorSubcoreMesh(core_axis_name="c", subcore_axis_name="s"),
        scratch_types=[
            pltpu.VMEM((_CHW,), jnp.int32),
            pltpu.VMEM((_CHW,), jnp.float32),
            pltpu.VMEM((_CHW,), jnp.float32),
            pltpu.VMEM((_CHW,), jnp.int32),
            pltpu.VMEM((_CHW,), jnp.float32),
            pltpu.VMEM((_CHW,), jnp.int32),
            pltpu.VMEM((_CHW,), jnp.int32),
            pltpu.VMEM((_CHW,), jnp.int32),
            pltpu.VMEM((_CHW, H), jnp.float32),
            pltpu.VMEM((_CHW, H), jnp.float32),
            pltpu.VMEM((16,), jnp.int32),
            pltpu.VMEM((16, H), jnp.float32),
            pltpu.SemaphoreType.DMA,
            pltpu.SemaphoreType.DMA,
            pltpu.SemaphoreType.DMA,
            pltpu.SemaphoreType.DMA,
            pltpu.VMEM((400,), jnp.float32),
        ],
    )


def _y_body(x_ref, sel_ref, w_ref, deg_ref, y_ref):
    dis = jax.lax.rsqrt(deg_ref[...])
    y_ref[...] = jnp.dot(x_ref[...] * sel_ref[...], w_ref[...],
                         preferred_element_type=jnp.float32) * dis


def _compute_y(x, sel, W, deg):
    return pl.pallas_call(
        _y_body,
        grid=(N // BLK,),
        in_specs=[
            pl.BlockSpec((BLK, H), lambda i: (i, 0)),
            pl.BlockSpec((BLK, 1), lambda i: (i, 0)),
            pl.BlockSpec((H, H), lambda i: (0, 0)),
            pl.BlockSpec((BLK, 1), lambda i: (i, 0)),
        ],
        out_specs=pl.BlockSpec((BLK, H), lambda i: (i, 0)),
        out_shape=jax.ShapeDtypeStruct((N, H), jnp.float32),
    )(x, sel, W, deg.reshape(N, 1))


def _conv_body(agg_ref, y_ref, deg_ref, b_ref, kept_ref, p_ref, pn_ref,
               xs_ref, sc_ref):
    i = pl.program_id(0)
    rows = i * BLKE + jax.lax.broadcasted_iota(jnp.int32, (BLKE, 1), 0)
    dis = jax.lax.rsqrt(deg_ref[...])
    conv = dis * (agg_ref[0] + agg_ref[1] + y_ref[...]) + b_ref[...]
    xl = jnp.where(kept_ref[...] != 0.0, jnp.maximum(conv, 0.0), 0.0)
    score = jnp.tanh(
        jnp.dot(xl, p_ref[...], preferred_element_type=jnp.float32)
        / pn_ref[0, 0])
    sc_ref[...] = score
    xs = xl * score
    pad = jnp.where(rows < _RZERO, -_BIG, 0.0)
    xs_ref[...] = jnp.where(rows < N, xs, pad)


def _conv_epilogue(parts, y, deg, b, keptf, p, pn):
    return pl.pallas_call(
        _conv_body,
        grid=(_NP // BLKE,),
        in_specs=[
            pl.BlockSpec((_NC, BLKE, H), lambda i: (0, i, 0)),
            pl.BlockSpec((BLKE, H), lambda i: (i, 0)),
            pl.BlockSpec((BLKE, 1), lambda i: (i, 0)),
            pl.BlockSpec((1, H), lambda i: (0, 0)),
            pl.BlockSpec((BLKE, 1), lambda i: (i, 0)),
            pl.BlockSpec((H, 1), lambda i: (0, 0)),
            pl.BlockSpec((1, 1), lambda i: (0, 0)),
        ],
        out_specs=[
            pl.BlockSpec((BLKE, H), lambda i: (i, 0)),
            pl.BlockSpec((BLKE, 1), lambda i: (i, 0)),
        ],
        out_shape=[
            jax.ShapeDtypeStruct((_NP, H), jnp.float32),
            jax.ShapeDtypeStruct((_NP, 1), jnp.float32),
        ],
    )(parts, y, deg.reshape(N, 1), b.reshape(1, H), keptf.reshape(N, 1),
      p.reshape(H, 1), pn.reshape(1, 1))


def kernel(x, edge_index, batch, W0, b0, p0, W1, b1, p1, W2, b2, p2):
    src = edge_index[0].astype(jnp.int32)
    dst = edge_index[1].astype(jnp.int32)
    batch = batch.astype(jnp.int32)
    params = [(W0, b0, p0), (W1, b1, p1), (W2, b2, p2)]

    src4 = src.reshape(_NC, _NS, _NCH // _IBK, _IBK, _CHW)
    dst4 = dst.reshape(_NC, _NS, _NCH // _IBK, _IBK, _CHW)

    node_off = jnp.searchsorted(
        batch, jnp.arange(G + 1, dtype=jnp.int32)).astype(jnp.int32)

    keptf = jnp.ones((N,), jnp.float32)
    selcol = jnp.ones((N, 1), jnp.float32)
    perm = jnp.arange(N, dtype=jnp.int32)
    cnts = node_off[1:] - node_off[:-1]
    xin = x
    out = None
    for W, b, p in params:
        hparts = _sc_hist_fn()(keptf, src4, dst4)
        deg = 1.0 + hparts[:N] + hparts[_NP:_NP + N]
        y = _compute_y(xin, selcol, W, deg)
        parts = _sc_aggregate_fn()(y, src4, dst4)
        xs_pad, score2 = _conv_epilogue(parts, y, deg, b, keptf, p,
                                        jnp.linalg.norm(p))
        score = score2[:N, 0]

        key = batch.astype(jnp.float32) * 4.0 - score
        kp = key[perm]
        _, order = lax.sort((kp, perm), num_keys=1, is_stable=True)
        order_pad = jnp.pad(order, (0, _NP - N))
        kept_s = keptf[order]
        cs = jnp.cumsum(kept_s)
        kept_s_pad = jnp.pad(kept_s, (0, _NP - N))
        cs_pad = jnp.pad(cs, (0, _NP - N))

        ks = jnp.ceil(RATIO * cnts.astype(jnp.float32)).astype(jnp.int32)
        new_off = jnp.cumsum(ks) - ks
        kept_off = jnp.cumsum(cnts) - cnts
        tab = jnp.zeros((400,), jnp.float32)
        tab = tab.at[:G + 1].set(node_off.astype(jnp.float32))
        tab = tab.at[80:80 + G].set(new_off.astype(jnp.float32))
        tab = tab.at[160:160 + G].set(kept_off.astype(jnp.float32))
        tab = tab.at[240:240 + G].set(ks.astype(jnp.float32))
        tab = tab.at[384].set(jnp.sum(ks).astype(jnp.float32))

        sel, perm_next, ms = _sc_pool_fn()(
            order_pad, kept_s_pad, cs_pad, xs_pad, tab)

        ksf = ks.astype(jnp.float32)
        mean = ms[72:72 + G] / jnp.maximum(ksf, 1.0)[:, None]
        r = jnp.concatenate([ms[:G], mean], axis=1)
        out = r if out is None else out + r

        keptf = sel[:N]
        selcol = keptf.reshape(N, 1)
        xin = xs_pad
        perm = perm_next[:N]
        cnts = ks
    return out


# speedup vs baseline: 14.2888x; 1.0743x over previous
import functools

import jax
import jax.numpy as jnp
from jax import lax
from jax.experimental import pallas as pl
from jax.experimental.pallas import tpu as pltpu
from jax.experimental.pallas import tpu_sc as plsc

N = 10000
E = 320000
H = 128
G = 64
RATIO = 0.5
BLK = 2000
BLKE = 2048

_NC = 2
_NS = 16
_CHW = 80
_NCH = E // (_NC * _NS * _CHW)
_IBK = 25
_NP = 10240
_RPT = _NP // _NS
_ZB = 64
_BIG = 1e30
_RMAX = 10000
_RZERO = 10016
_DUMP = 10200


def _hist_body(keptf_hbm, src_hbm, dst_hbm, out_hbm, sidx, didx, vbuf, zbuf,
               sem, semB, acc):
    c = lax.axis_index("c")
    s = lax.axis_index("s")

    for i in range(_RPT // 16):
        zbuf[pl.ds(i * 16, 16)] = jnp.zeros((16,), jnp.float32)
    pltpu.sync_copy(zbuf, acc.at[pl.ds(s * _RPT, _RPT)])
    plsc.subcore_barrier()

    vb0 = vbuf.at[0]
    vb1 = vbuf.at[1]

    def block(b, _):
        pltpu.sync_copy(src_hbm.at[c, s, b], sidx)
        pltpu.sync_copy(dst_hbm.at[c, s, b], didx)
        pltpu.async_copy(keptf_hbm.at[sidx.at[0]], vb0, sem)

        def pair(t, _):
            i = 2 * t
            pltpu.async_copy(keptf_hbm.at[sidx.at[i + 1]], vb1, semB)
            pltpu.make_async_copy(keptf_hbm.at[sidx.at[i]], vb0, sem).wait()
            pltpu.sync_copy(vb0, acc.at[didx.at[i]], add=True)
            pltpu.async_copy(keptf_hbm.at[sidx.at[i + 2]], vb0, sem)
            pltpu.make_async_copy(
                keptf_hbm.at[sidx.at[i + 1]], vb1, semB).wait()
            pltpu.sync_copy(vb1, acc.at[didx.at[i + 1]], add=True)
            return 0
        lax.fori_loop(0, (_IBK - 1) // 2, pair, 0, unroll=False)
        pltpu.make_async_copy(keptf_hbm.at[sidx.at[_IBK - 1]], vb0, sem).wait()
        pltpu.sync_copy(vb0, acc.at[didx.at[_IBK - 1]], add=True)
        return 0
    lax.fori_loop(0, _NCH // _IBK, block, 0, unroll=False)
    plsc.subcore_barrier()
    pltpu.sync_copy(acc.at[pl.ds(s * _RPT, _RPT)],
                    out_hbm.at[pl.ds(c * _NP + s * _RPT, _RPT)])


@functools.cache
def _sc_hist_fn():
    return pl.kernel(
        _hist_body,
        out_type=jax.ShapeDtypeStruct((_NC * _NP,), jnp.float32),
        mesh=plsc.VectorSubcoreMesh(core_axis_name="c", subcore_axis_name="s"),
        scratch_types=[
            pltpu.VMEM((_IBK, _CHW), jnp.int32),
            pltpu.VMEM((_IBK, _CHW), jnp.int32),
            pltpu.VMEM((2, _CHW), jnp.float32),
            pltpu.VMEM((_RPT,), jnp.float32),
            pltpu.SemaphoreType.DMA,
            pltpu.SemaphoreType.DMA,
            pltpu.VMEM_SHARED((_NP,), jnp.float32),
        ],
    )


def _agg_body(y_hbm, src_hbm, dst_hbm, out_hbm, sidx, didx, rbuf, zbuf, sem,
              semB, acc):
    c = lax.axis_index("c")
    s = lax.axis_index("s")

    def zrow(i, _):
        for j in range(H // 16):
            zbuf[i, pl.ds(j * 16, 16)] = jnp.zeros((16,), jnp.float32)
        return 0
    lax.fori_loop(0, _ZB, zrow, 0, unroll=False)
    for k in range(_RPT // _ZB):
        pltpu.sync_copy(zbuf, acc.at[pl.ds(s * _RPT + k * _ZB, _ZB)])
    plsc.subcore_barrier()

    rb0 = rbuf.at[0]
    rb1 = rbuf.at[1]

    def block(b, _):
        pltpu.sync_copy(src_hbm.at[c, s, b], sidx)
        pltpu.sync_copy(dst_hbm.at[c, s, b], didx)
        pltpu.async_copy(y_hbm.at[sidx.at[0]], rb0, sem)

        def pair(t, _):
            i = 2 * t
            pltpu.async_copy(y_hbm.at[sidx.at[i + 1]], rb1, semB)
            pltpu.make_async_copy(y_hbm.at[sidx.at[i]], rb0, sem).wait()
            pltpu.sync_copy(rb0, acc.at[didx.at[i]], add=True)
            pltpu.async_copy(y_hbm.at[sidx.at[i + 2]], rb0, sem)
            pltpu.make_async_copy(y_hbm.at[sidx.at[i + 1]], rb1, semB).wait()
            pltpu.sync_copy(rb1, acc.at[didx.at[i + 1]], add=True)
            return 0
        lax.fori_loop(0, (_IBK - 1) // 2, pair, 0, unroll=False)
        pltpu.make_async_copy(y_hbm.at[sidx.at[_IBK - 1]], rb0, sem).wait()
        pltpu.sync_copy(rb0, acc.at[didx.at[_IBK - 1]], add=True)
        return 0
    lax.fori_loop(0, _NCH // _IBK, block, 0, unroll=False)
    plsc.subcore_barrier()
    pltpu.sync_copy(acc.at[pl.ds(s * _RPT, _RPT)],
                    out_hbm.at[c, pl.ds(s * _RPT, _RPT)])


@functools.cache
def _sc_aggregate_fn():
    return pl.kernel(
        _agg_body,
        out_type=jax.ShapeDtypeStruct((_NC, _NP, H), jnp.float32),
        mesh=plsc.VectorSubcoreMesh(core_axis_name="c", subcore_axis_name="s"),
        scratch_types=[
            pltpu.VMEM((_IBK, _CHW), jnp.int32),
            pltpu.VMEM((_IBK, _CHW), jnp.int32),
            pltpu.VMEM((2, _CHW, H), jnp.float32),
            pltpu.VMEM((_ZB, H), jnp.float32),
            pltpu.SemaphoreType.DMA,
            pltpu.SemaphoreType.DMA,
            pltpu.VMEM_SHARED((_NP, H), jnp.float32),
        ],
    )


def _pool_body(order_hbm, ks_s_hbm, cs_hbm, xs_hbm, tab_hbm,
               sel_hbm, perm_hbm, ms_hbm,
               obuf, kbuf, csb, tbuf, wbuf, pbuf, m1buf, m2buf, rbufA, rbufB,
               idx16, out16, sem, semB, semS, semP, tab):
    w = lax.axis_index("c") * _NS + lax.axis_index("s")
    iota16 = lax.iota(jnp.int32, 16)
    zero16 = jnp.zeros((16,), jnp.float32)
    nbig16 = jnp.full((16,), -_BIG, jnp.float32)

    pltpu.sync_copy(tab_hbm, tab)

    def sread(i):
        return tab[pl.ds(i, 16)][0]

    total = sread(4 * 80 + G).astype(jnp.int32)

    for gl in range(2):
        g = w * 2 + gl
        off = sread(g).astype(jnp.int32)
        off1 = sread(g + 1).astype(jnp.int32)
        ko = sread(80 + g).astype(jnp.int32)
        koff = sread(2 * 80 + g)
        ksgf = sread(3 * 80 + g)
        tail0 = total + (off - ko)
        base = (off // 8) * 8
        ntrip = (off1 - base + _CHW - 1) // _CHW

        def chunk2(t, accs):
            cbase = base + t * _CHW
            cpO = pltpu.async_copy(order_hbm.at[pl.ds(cbase, _CHW)], obuf,
                                   semS)
            cpK = pltpu.async_copy(ks_s_hbm.at[pl.ds(cbase, _CHW)], kbuf,
                                   semP)
            cpC = pltpu.async_copy(cs_hbm.at[pl.ds(cbase, _CHW)], csb, semB)
            cpO.wait()
            cpK.wait()
            cpC.wait()
            for j in range(_CHW // 16):
                posv = cbase + j * 16 + iota16
                mask = (posv >= off) & (posv < off1)
                kf = kbuf[pl.ds(j * 16, 16)]
                rank = csb[pl.ds(j * 16, 16)] - koff
                selv = mask & (kf > 0.0) & (rank <= ksgf)
                ov = obuf[pl.ds(j * 16, 16)]
                wbuf[pl.ds(j * 16, 16)] = jnp.where(selv, 1.0, 0.0)
                tbuf[pl.ds(j * 16, 16)] = jnp.where(mask, ov, _DUMP)
                inseg = (posv - off + 1).astype(jnp.float32)
                nonsel = inseg - jnp.minimum(jnp.maximum(rank, 0.0), ksgf)
                slot = jnp.where(
                    selv, ko + rank.astype(jnp.int32) - 1,
                    tail0 + nonsel.astype(jnp.int32) - 1)
                pbuf[pl.ds(j * 16, 16)] = jnp.where(mask, slot, _DUMP)
                m1buf[pl.ds(j * 16, 16)] = jnp.where(
                    selv, ov, _RMAX + (iota16 & 7))
                m2buf[pl.ds(j * 16, 16)] = jnp.where(
                    selv, ov, _RZERO + (iota16 & 7))
            cpS = pltpu.async_copy(wbuf, sel_hbm.at[tbuf], semS)
            cpP = pltpu.async_copy(obuf, perm_hbm.at[pbuf], semP)
            cpA = pltpu.async_copy(xs_hbm.at[m1buf], rbufA, sem)
            cpB = pltpu.async_copy(xs_hbm.at[m2buf], rbufB, semB)
            cpA.wait()
            cpB.wait()

            def row(r, a):
                mx = tuple(
                    jnp.maximum(a[j], rbufA[r, pl.ds(j * 16, 16)])
                    for j in range(8))
                sm = tuple(
                    a[8 + j] + rbufB[r, pl.ds(j * 16, 16)]
                    for j in range(8))
                return mx + sm
            accs = lax.fori_loop(0, _CHW, row, accs, unroll=False)
            cpS.wait()
            cpP.wait()
            return accs
        accs0 = tuple([nbig16] * 8 + [zero16] * 8)
        accs = lax.fori_loop(0, ntrip, chunk2, accs0, unroll=False)

        for j in range(H // 16):
            out16[0, pl.ds(j * 16, 16)] = accs[j]
            out16[1, pl.ds(j * 16, 16)] = accs[8 + j]
        idx16[pl.ds(0, 16)] = jnp.where(
            iota16 == 0, g,
            jnp.where(iota16 == 1, 72 + g, 136 + (iota16 & 7)))
        pltpu.sync_copy(out16, ms_hbm.at[idx16])


@functools.cache
def _sc_pool_fn():
    return pl.kernel(
        _pool_body,
        out_type=(
            jax.ShapeDtypeStruct((_NP,), jnp.float32),
            jax.ShapeDtypeStruct((_NP,), jnp.int32),
            jax.ShapeDtypeStruct((144, H), jnp.float32),
        ),
        mesh=plsc.VectorSubcoreMesh(core_axis_name="c", subcore_axis_name="s"),
        scratch_types=[
            pltpu.VMEM((_CHW,), jnp.int32),
            pltpu.VMEM((_CHW,), jnp.float32),
            pltpu.VMEM((_CHW,), jnp.float32),
            pltpu.VMEM((_CHW,), jnp.int32),
            pltpu.VMEM((_CHW,), jnp.float32),
            pltpu.VMEM((_CHW,), jnp.int32),
            pltpu.VMEM((_CHW,), jnp.int32),
            pltpu.VMEM((_CHW,), jnp.int32),
            pltpu.VMEM((_CHW, H), jnp.float32),
            pltpu.VMEM((_CHW, H), jnp.float32),
            pltpu.VMEM((16,), jnp.int32),
            pltpu.VMEM((16, H), jnp.float32),
            pltpu.SemaphoreType.DMA,
            pltpu.SemaphoreType.DMA,
            pltpu.SemaphoreType.DMA,
            pltpu.SemaphoreType.DMA,
            pltpu.VMEM((400,), jnp.float32),
        ],
    )


def _y_body(x_ref, sel_ref, w_ref, deg_ref, y_ref):
    dis = jax.lax.rsqrt(deg_ref[...])
    y_ref[...] = jnp.dot(x_ref[...] * sel_ref[...], w_ref[...],
                         preferred_element_type=jnp.float32) * dis


def _compute_y(x, sel, W, deg):
    return pl.pallas_call(
        _y_body,
        grid=(N // BLK,),
        in_specs=[
            pl.BlockSpec((BLK, H), lambda i: (i, 0)),
            pl.BlockSpec((BLK, 1), lambda i: (i, 0)),
            pl.BlockSpec((H, H), lambda i: (0, 0)),
            pl.BlockSpec((BLK, 1), lambda i: (i, 0)),
        ],
        out_specs=pl.BlockSpec((BLK, H), lambda i: (i, 0)),
        out_shape=jax.ShapeDtypeStruct((N, H), jnp.float32),
    )(x, sel, W, deg.reshape(N, 1))


def _conv_body(agg_ref, y_ref, deg_ref, b_ref, kept_ref, p_ref, pn_ref,
               xs_ref, sc_ref):
    i = pl.program_id(0)
    rows = i * BLKE + jax.lax.broadcasted_iota(jnp.int32, (BLKE, 1), 0)
    dis = jax.lax.rsqrt(deg_ref[...])
    conv = dis * (agg_ref[0] + agg_ref[1] + y_ref[...]) + b_ref[...]
    xl = jnp.where(kept_ref[...] != 0.0, jnp.maximum(conv, 0.0), 0.0)
    score = jnp.tanh(
        jnp.dot(xl, p_ref[...], preferred_element_type=jnp.float32)
        / pn_ref[0, 0])
    sc_ref[...] = score
    xs = xl * score
    pad = jnp.where(rows < _RZERO, -_BIG, 0.0)
    xs_ref[...] = jnp.where(rows < N, xs, pad)


def _conv_epilogue(parts, y, deg, b, keptf, p, pn):
    return pl.pallas_call(
        _conv_body,
        grid=(_NP // BLKE,),
        in_specs=[
            pl.BlockSpec((_NC, BLKE, H), lambda i: (0, i, 0)),
            pl.BlockSpec((BLKE, H), lambda i: (i, 0)),
            pl.BlockSpec((BLKE, 1), lambda i: (i, 0)),
            pl.BlockSpec((1, H), lambda i: (0, 0)),
            pl.BlockSpec((BLKE, 1), lambda i: (i, 0)),
            pl.BlockSpec((H, 1), lambda i: (0, 0)),
            pl.BlockSpec((1, 1), lambda i: (0, 0)),
        ],
        out_specs=[
            pl.BlockSpec((BLKE, H), lambda i: (i, 0)),
            pl.BlockSpec((BLKE, 1), lambda i: (i, 0)),
        ],
        out_shape=[
            jax.ShapeDtypeStruct((_NP, H), jnp.float32),
            jax.ShapeDtypeStruct((_NP, 1), jnp.float32),
        ],
    )(parts, y, deg.reshape(N, 1), b.reshape(1, H), keptf.reshape(N, 1),
      p.reshape(H, 1), pn.reshape(1, 1))


def kernel(x, edge_index, batch, W0, b0, p0, W1, b1, p1, W2, b2, p2):
    src = edge_index[0].astype(jnp.int32)
    dst = edge_index[1].astype(jnp.int32)
    batch = batch.astype(jnp.int32)
    params = [(W0, b0, p0), (W1, b1, p1), (W2, b2, p2)]

    src4 = src.reshape(_NC, _NS, _NCH // _IBK, _IBK, _CHW)
    dst4 = dst.reshape(_NC, _NS, _NCH // _IBK, _IBK, _CHW)

    node_off = jnp.searchsorted(
        batch, jnp.arange(G + 1, dtype=jnp.int32)).astype(jnp.int32)

    keptf = jnp.ones((N,), jnp.float32)
    selcol = jnp.ones((N, 1), jnp.float32)
    perm = jnp.arange(N, dtype=jnp.int32)
    cnts = node_off[1:] - node_off[:-1]
    xin = x
    out = None
    for W, b, p in params:
        hparts = _sc_hist_fn()(keptf, src4, dst4)
        deg = 1.0 + hparts[:N] + hparts[_NP:_NP + N]
        y = _compute_y(xin, selcol, W, deg)
        parts = _sc_aggregate_fn()(y, src4, dst4)
        xs_pad, score2 = _conv_epilogue(parts, y, deg, b, keptf, p,
                                        jnp.linalg.norm(p))
        score = score2[:N, 0]

        key = batch.astype(jnp.float32) * 4.0 - score
        kp = key[perm]
        _, order = lax.sort((kp, perm), num_keys=1, is_stable=True)
        order_pad = jnp.pad(order, (0, _NP - N))
        kept_s = keptf[order]
        cs = jnp.cumsum(kept_s)
        kept_s_pad = jnp.pad(kept_s, (0, _NP - N))
        cs_pad = jnp.pad(cs, (0, _NP - N))

        ks = jnp.ceil(RATIO * cnts.astype(jnp.float32)).astype(jnp.int32)
        new_off = jnp.cumsum(ks) - ks
        kept_off = jnp.cumsum(cnts) - cnts
        tab = jnp.zeros((400,), jnp.float32)
        tab = tab.at[:G + 1].set(node_off.astype(jnp.float32))
        tab = tab.at[80:80 + G].set(new_off.astype(jnp.float32))
        tab = tab.at[160:160 + G].set(kept_off.astype(jnp.float32))
        tab = tab.at[240:240 + G].set(ks.astype(jnp.float32))
        tab = tab.at[384].set(jnp.sum(ks).astype(jnp.float32))

        sel, perm_next, ms = _sc_pool_fn()(
            order_pad, kept_s_pad, cs_pad, xs_pad, tab)

        ksf = ks.astype(jnp.float32)
        mean = ms[72:72 + G] / jnp.maximum(ksf, 1.0)[:, None]
        r = jnp.concatenate([ms[:G], mean], axis=1)
        out = r if out is None else out + r

        keptf = sel[:N]
        selcol = keptf.reshape(N, 1)
        xin = xs_pad
        perm = perm_next[:N]
        cnts = ks
    return out
